# speedup vs baseline: 2.5419x; 2.5419x over previous
import functools

import jax
import jax.numpy as jnp
from jax.experimental import pallas as pl
from jax.experimental.pallas import tpu as pltpu

_N_EXPERTS = 8
_N_ROUTED = 7
_TOP_K = 2
_LANES = 128


def _gelu(v):
    return 0.5 * v * (1.0 + jax.lax.erf(v * 0.7071067811865476))


def _moe_body(x_ref, wg_ref, wfc_ref, bfc_ref, wproj_ref, bproj_ref, o_ref):
    xt = x_ref[...]
    xb = xt.astype(jnp.bfloat16)

    logits = jax.lax.dot_general(
        xt, wg_ref[...], (((1,), (0,)), ((), ())),
        preferred_element_type=jnp.float32)
    lane = jax.lax.broadcasted_iota(jnp.int32, logits.shape, 1)
    neg = jnp.float32(-1e30)
    logits = jnp.where(lane < _N_ROUTED, logits, neg)

    m0 = jnp.max(logits, axis=1, keepdims=True)
    idx0 = jnp.min(jnp.where(logits == m0, lane, _LANES), axis=1,
                   keepdims=True)
    logits1 = jnp.where(lane == idx0, neg, logits)
    m1 = jnp.max(logits1, axis=1, keepdims=True)
    idx1 = jnp.min(jnp.where(logits1 == m1, lane, _LANES), axis=1,
                   keepdims=True)
    e1 = jnp.exp(m1 - m0)
    w0 = 1.0 / (1.0 + e1)
    w1 = e1 * w0

    def ffn(j):
        h = jax.lax.dot_general(
            xb, wfc_ref[j], (((1,), (0,)), ((), ())),
            preferred_element_type=jnp.float32)
        h = _gelu(h + bfc_ref[j, :])
        y = jax.lax.dot_general(
            h.astype(jnp.bfloat16), wproj_ref[j], (((1,), (0,)), ((), ())),
            preferred_element_type=jnp.float32)
        return y + bproj_ref[j, :]

    acc = ffn(0)
    for j in range(_N_ROUTED):
        gate = w0 * (idx0 == j) + w1 * (idx1 == j)
        acc = acc + gate * ffn(j + 1)
    o_ref[...] = acc


@jax.jit
def kernel(x, Ws_fc, bs_fc, Ws_proj, bs_proj, Wr_fc, br_fc, Wr_proj, br_proj, Wg):
    B, S, D = x.shape
    N = B * S
    HID = Ws_fc.shape[1]
    xf = x.reshape(N, D)

    wfc = jnp.concatenate([Ws_fc[None], Wr_fc], axis=0).astype(jnp.bfloat16)
    wproj = jnp.concatenate([Ws_proj[None], Wr_proj], axis=0).astype(jnp.bfloat16)
    bfc = jnp.concatenate([bs_fc[None], br_fc], axis=0)
    bproj = jnp.concatenate([bs_proj[None], br_proj], axis=0)
    wg = jnp.pad(Wg, ((0, 0), (0, _LANES - Wg.shape[1])))

    T = 512
    grid = (N // T,)
    out = pl.pallas_call(
        _moe_body,
        grid=grid,
        in_specs=[
            pl.BlockSpec((T, D), lambda i: (i, 0)),
            pl.BlockSpec((D, _LANES), lambda i: (0, 0)),
            pl.BlockSpec((_N_EXPERTS, D, HID), lambda i: (0, 0, 0)),
            pl.BlockSpec((_N_EXPERTS, HID), lambda i: (0, 0)),
            pl.BlockSpec((_N_EXPERTS, HID, D), lambda i: (0, 0, 0)),
            pl.BlockSpec((_N_EXPERTS, D), lambda i: (0, 0)),
        ],
        out_specs=pl.BlockSpec((T, D), lambda i: (i, 0)),
        out_shape=jax.ShapeDtypeStruct((N, D), jnp.float32),
    )(xf, wg, wfc, bfc, wproj, bproj)
    return out.reshape(B, S, D)
